# baseline (speedup 1.0000x reference)
import jax
import jax.numpy as jnp
from jax import lax
from jax.experimental import pallas as pl
from jax.experimental.pallas import tpu as pltpu
from jax.experimental.pallas import tpu_sc as plsc

BATCH = 8192
WIDTH = 2048
LANES = 16
NUM_CORES = 2
NUM_SUBCORES = 16
NUM_TILES = NUM_CORES * NUM_SUBCORES
ROWS_PER_TILE = BATCH // NUM_TILES
CHUNK_ROWS = 8
NUM_CHUNKS = ROWS_PER_TILE // CHUNK_ROWS
NBUF_IN = 2
NBUF_OUT = 4


def _sc_body(x_hbm, perm_hbm, y_hbm, perm_v, *bufs):
    ins = bufs[:NBUF_IN]
    outs = bufs[NBUF_IN:NBUF_IN + NBUF_OUT]
    sins = bufs[NBUF_IN + NBUF_OUT:NBUF_IN + NBUF_OUT + NBUF_IN]
    souts = bufs[NBUF_IN + NBUF_OUT + NBUF_IN:]

    wid = lax.axis_index("s") * NUM_CORES + lax.axis_index("c")
    row_base = wid * ROWS_PER_TILE
    pltpu.sync_copy(perm_hbm, perm_v)

    def in_slice(c):
        return x_hbm.at[pl.ds(row_base + c * CHUNK_ROWS, CHUNK_ROWS)]

    def out_slice(c):
        return y_hbm.at[pl.ds(row_base + c * CHUNK_ROWS, CHUNK_ROWS)]

    def permute_chunk(in_ref, out_ref):
        @plsc.parallel_loop(0, WIDTH // LANES, unroll=8)
        def _col_loop(jc):
            cv = perm_v[pl.ds(jc * LANES, LANES)]
            off = jc * LANES
            for r in range(CHUNK_ROWS):
                rv = jnp.full((LANES,), r, jnp.int32)
                vals = plsc.load_gather(in_ref, [rv, cv])
                out_ref[r, pl.ds(off, LANES)] = vals

    for b in range(NBUF_IN - 1):
        pltpu.async_copy(in_slice(b), ins[b], sins[b])

    step = max(NBUF_IN, NBUF_OUT)

    @pl.loop(0, NUM_CHUNKS, step=step)
    def _chunk_loop(c):
        for b in range(step):
            cc = c + b
            ib = b % NBUF_IN
            ob = b % NBUF_OUT
            pf = cc + NBUF_IN - 1
            pb = (b + NBUF_IN - 1) % NBUF_IN

            @pl.when(pf < NUM_CHUNKS)
            def _prefetch():
                pltpu.async_copy(in_slice(pf), ins[pb], sins[pb])

            pltpu.make_async_copy(in_slice(cc), ins[ib], sins[ib]).wait()

            @pl.when(cc >= NBUF_OUT)
            def _drain_out():
                pltpu.make_async_copy(outs[ob], out_slice(cc), souts[ob]).wait()

            permute_chunk(ins[ib], outs[ob])
            pltpu.async_copy(outs[ob], out_slice(cc), souts[ob])

    for k in range(NUM_CHUNKS - NBUF_OUT, NUM_CHUNKS):
        ob = (k % step) % NBUF_OUT
        pltpu.make_async_copy(outs[ob], out_slice(k), souts[ob]).wait()


def kernel(x, perm):
    mesh = plsc.VectorSubcoreMesh(
        core_axis_name="c", subcore_axis_name="s",
        num_cores=NUM_CORES, num_subcores=NUM_SUBCORES)
    run = pl.kernel(
        _sc_body,
        out_type=jax.ShapeDtypeStruct((BATCH, WIDTH), jnp.float32),
        mesh=mesh,
        scratch_types=(
            [pltpu.VMEM((WIDTH,), jnp.int32)]
            + [pltpu.VMEM((CHUNK_ROWS, WIDTH), jnp.float32)] * (NBUF_IN + NBUF_OUT)
            + [pltpu.SemaphoreType.DMA] * (NBUF_IN + NBUF_OUT)
        ),
        compiler_params=pltpu.CompilerParams(
            needs_layout_passes=False, use_tc_tiling_on_sc=True),
    )
    y = run(x, perm.astype(jnp.int32))
    return (y, 0.0)


# speedup vs baseline: 1.0350x; 1.0350x over previous
import jax
import jax.numpy as jnp
from jax import lax
from jax.experimental import pallas as pl
from jax.experimental.pallas import tpu as pltpu
from jax.experimental.pallas import tpu_sc as plsc

BATCH = 8192
WIDTH = 2048
LANES = 16
NUM_CORES = 2
NUM_SUBCORES = 16
NUM_TILES = NUM_CORES * NUM_SUBCORES
ROWS_PER_TILE = BATCH // NUM_TILES
CHUNK_ROWS = 8
NUM_CHUNKS = ROWS_PER_TILE // CHUNK_ROWS
NBUF_IN = 4
NBUF_OUT = 2


def _sc_body(x_hbm, perm_hbm, y_hbm, perm_v, *bufs):
    ins = bufs[:NBUF_IN]
    outs = bufs[NBUF_IN:NBUF_IN + NBUF_OUT]
    sins = bufs[NBUF_IN + NBUF_OUT:NBUF_IN + NBUF_OUT + NBUF_IN]
    souts = bufs[NBUF_IN + NBUF_OUT + NBUF_IN:]

    wid = lax.axis_index("s") * NUM_CORES + lax.axis_index("c")
    row_base = wid * ROWS_PER_TILE
    pltpu.sync_copy(perm_hbm, perm_v)

    def in_slice(c):
        return x_hbm.at[pl.ds(row_base + c * CHUNK_ROWS, CHUNK_ROWS)]

    def out_slice(c):
        return y_hbm.at[pl.ds(row_base + c * CHUNK_ROWS, CHUNK_ROWS)]

    def permute_chunk(in_ref, out_ref):
        @plsc.parallel_loop(0, WIDTH // LANES, unroll=8)
        def _col_loop(jc):
            cv = perm_v[pl.ds(jc * LANES, LANES)]
            off = jc * LANES
            for r in range(CHUNK_ROWS):
                rv = jnp.full((LANES,), r, jnp.int32)
                vals = plsc.load_gather(in_ref, [rv, cv])
                out_ref[r, pl.ds(off, LANES)] = vals

    for b in range(NBUF_IN - 1):
        pltpu.async_copy(in_slice(b), ins[b], sins[b])

    step = max(NBUF_IN, NBUF_OUT)

    @pl.loop(0, NUM_CHUNKS, step=step)
    def _chunk_loop(c):
        for b in range(step):
            cc = c + b
            ib = b % NBUF_IN
            ob = b % NBUF_OUT
            pf = cc + NBUF_IN - 1
            pb = (b + NBUF_IN - 1) % NBUF_IN

            @pl.when(pf < NUM_CHUNKS)
            def _prefetch():
                pltpu.async_copy(in_slice(pf), ins[pb], sins[pb])

            @pl.when(cc >= NBUF_OUT)
            def _drain_out():
                pltpu.make_async_copy(outs[ob], out_slice(cc), souts[ob]).wait()

            pltpu.make_async_copy(in_slice(cc), ins[ib], sins[ib]).wait()
            permute_chunk(ins[ib], outs[ob])
            pltpu.async_copy(outs[ob], out_slice(cc), souts[ob])

    for k in range(NUM_CHUNKS - NBUF_OUT, NUM_CHUNKS):
        ob = (k % step) % NBUF_OUT
        pltpu.make_async_copy(outs[ob], out_slice(k), souts[ob]).wait()


def kernel(x, perm):
    mesh = plsc.VectorSubcoreMesh(
        core_axis_name="c", subcore_axis_name="s",
        num_cores=NUM_CORES, num_subcores=NUM_SUBCORES)
    run = pl.kernel(
        _sc_body,
        out_type=jax.ShapeDtypeStruct((BATCH, WIDTH), jnp.float32),
        mesh=mesh,
        scratch_types=(
            [pltpu.VMEM((WIDTH,), jnp.int32)]
            + [pltpu.VMEM((CHUNK_ROWS, WIDTH), jnp.float32)] * (NBUF_IN + NBUF_OUT)
            + [pltpu.SemaphoreType.DMA] * (NBUF_IN + NBUF_OUT)
        ),
        compiler_params=pltpu.CompilerParams(
            needs_layout_passes=False, use_tc_tiling_on_sc=True),
    )
    y = run(x, perm.astype(jnp.int32))
    return (y, 0.0)
